# baseline (speedup 1.0000x reference)
import functools

import jax
import jax.numpy as jnp
from jax import lax
from jax.experimental import pallas as pl
from jax.experimental.pallas import tpu as pltpu
from jax.experimental.pallas import tpu_sc as plsc


def _make_sc_gather(V, D, B):
    info = plsc.get_sparse_core_info()
    NC, NS = info.num_cores, info.num_subcores
    NW = NC * NS
    assert B % (8 * NW) == 0
    b_per_w = B // NW
    mesh = plsc.VectorSubcoreMesh(core_axis_name="c", subcore_axis_name="s")

    @functools.partial(
        pl.kernel,
        mesh=mesh,
        out_type=jax.ShapeDtypeStruct((B, D), jnp.float32),
        scratch_types=[
            pltpu.VMEM((b_per_w,), jnp.int32),
            pltpu.VMEM((b_per_w, D), jnp.float32),
            pltpu.SemaphoreType.DMA,
        ],
        compiler_params=pltpu.CompilerParams(use_tc_tiling_on_sc=False),
    )
    def gather_k(table_hbm, idx_hbm, out_hbm, idx_v, rows_v, sem):
        wid = lax.axis_index("s") * NC + lax.axis_index("c")
        base = wid * b_per_w
        pltpu.sync_copy(idx_hbm.at[pl.ds(base, b_per_w)], idx_v)
        pltpu.async_copy(table_hbm.at[idx_v], rows_v, sem).wait()
        pltpu.sync_copy(rows_v, out_hbm.at[pl.ds(base, b_per_w)])

    return gather_k


def _proj_body(e_ref, w_ref, b_ref, o_ref):
    o_ref[...] = (
        jnp.dot(e_ref[...], w_ref[...], preferred_element_type=jnp.float32)
        + b_ref[...]
    )


def _make_tc_proj(B, D, blk):
    return pl.pallas_call(
        _proj_body,
        grid=(B // blk,),
        in_specs=[
            pl.BlockSpec((blk, D), lambda i: (i, 0)),
            pl.BlockSpec((D, D), lambda i: (0, 0)),
            pl.BlockSpec((1, D), lambda i: (0, 0)),
        ],
        out_specs=pl.BlockSpec((blk, D), lambda i: (i, 0)),
        out_shape=jax.ShapeDtypeStruct((B, D), jnp.float32),
    )


def kernel(y, embed_table, W, b):
    B = y.shape[0]
    V, D = embed_table.shape
    e = _make_sc_gather(V, D, B)(embed_table, y.astype(jnp.int32))
    return _make_tc_proj(B, D, blk=2048)(e, W, b.reshape(1, D))


# speedup vs baseline: 1.8063x; 1.8063x over previous
import functools

import jax
import jax.numpy as jnp
from jax import lax
from jax.experimental import pallas as pl
from jax.experimental.pallas import tpu as pltpu
from jax.experimental.pallas import tpu_sc as plsc


def _make_sc_gather_t(D, V, B):
    info = plsc.get_sparse_core_info()
    NC, NS, L = info.num_cores, info.num_subcores, info.num_lanes
    NW = NC * NS
    assert D % NW == 0
    rows_per_w = D // NW
    B_CH = 8192 if B % 8192 == 0 else B
    n_ch = B // B_CH
    mesh = plsc.VectorSubcoreMesh(core_axis_name="c", subcore_axis_name="s")

    @functools.partial(
        pl.kernel,
        mesh=mesh,
        out_type=jax.ShapeDtypeStruct((D, B), jnp.float32),
        scratch_types=[
            pltpu.VMEM((V,), jnp.float32),
            pltpu.VMEM((B,), jnp.int32),
            pltpu.VMEM((B_CH,), jnp.float32),
        ],
        compiler_params=pltpu.CompilerParams(needs_layout_passes=False),
    )
    def gather_k(tbl_hbm, idx_hbm, out_hbm, row_v, idx_v, out_v):
        wid = lax.axis_index("s") * NC + lax.axis_index("c")
        pltpu.sync_copy(idx_hbm, idx_v)
        for rr in range(rows_per_w):
            d = wid * rows_per_w + rr
            pltpu.sync_copy(tbl_hbm.at[d], row_v)
            for c in range(n_ch):
                def body(j, carry, _c=c):
                    idxs = idx_v[pl.ds(_c * B_CH + j * L, L)]
                    out_v[pl.ds(j * L, L)] = plsc.load_gather(row_v, [idxs])
                    return carry

                lax.fori_loop(0, B_CH // L, body, 0, unroll=4)
                pltpu.sync_copy(out_v, out_hbm.at[d, pl.ds(c * B_CH, B_CH)])

    return gather_k


def _proj_t_body(w_ref, e_ref, b_ref, o_ref):
    o_ref[...] = (
        lax.dot_general(
            w_ref[...],
            e_ref[...],
            (((0,), (0,)), ((), ())),
            preferred_element_type=jnp.float32,
        )
        + b_ref[...]
    )


def _make_tc_proj_t(D, B, blk):
    return pl.pallas_call(
        _proj_t_body,
        grid=(B // blk,),
        in_specs=[
            pl.BlockSpec((D, D), lambda i: (0, 0)),
            pl.BlockSpec((D, blk), lambda i: (0, i)),
            pl.BlockSpec((D, 1), lambda i: (0, 0)),
        ],
        out_specs=pl.BlockSpec((D, blk), lambda i: (0, i)),
        out_shape=jax.ShapeDtypeStruct((D, B), jnp.float32),
    )


def kernel(y, embed_table, W, b):
    V, D = embed_table.shape
    B = y.shape[0]
    tbl_t = embed_table.T
    e_t = _make_sc_gather_t(D, V, B)(tbl_t, y.astype(jnp.int32))
    out_t = _make_tc_proj_t(D, B, blk=2048)(W, e_t, b.reshape(D, 1))
    return out_t.T


# speedup vs baseline: 1.9348x; 1.0712x over previous
import functools

import jax
import jax.numpy as jnp
from jax import lax
from jax.experimental import pallas as pl
from jax.experimental.pallas import tpu as pltpu
from jax.experimental.pallas import tpu_sc as plsc


def _make_sc_gather_t(D, V, B):
    info = plsc.get_sparse_core_info()
    NC, NS, L = info.num_cores, info.num_subcores, info.num_lanes
    NW = NC * NS
    assert D % NW == 0
    rows_per_w = D // NW
    B_CH = 4096 if B % 4096 == 0 else B
    n_ch = B // B_CH
    mesh = plsc.VectorSubcoreMesh(core_axis_name="c", subcore_axis_name="s")

    @functools.partial(
        pl.kernel,
        mesh=mesh,
        out_type=jax.ShapeDtypeStruct((D, B), jnp.float32),
        scratch_types=[
            pltpu.VMEM((V,), jnp.float32),
            pltpu.VMEM((B,), jnp.int32),
            pltpu.VMEM((B_CH,), jnp.float32),
            pltpu.VMEM((B_CH,), jnp.float32),
            pltpu.SemaphoreType.DMA,
            pltpu.SemaphoreType.DMA,
            pltpu.SemaphoreType.DMA,
            pltpu.SemaphoreType.DMA,
        ],
        compiler_params=pltpu.CompilerParams(needs_layout_passes=False),
    )
    def gather_k(tbl_hbm, idx_hbm, out_hbm, row_v, idx_v, out_v0, out_v1,
                 sem_i, sem_r, sem_o0, sem_o1):
        wid = lax.axis_index("s") * NC + lax.axis_index("c")
        out_bufs = (out_v0, out_v1)
        out_sems = (sem_o0, sem_o1)
        pending = [None, None]
        ci = pltpu.async_copy(idx_hbm, idx_v, sem_i)
        for rr in range(rows_per_w):
            d = wid * rows_per_w + rr
            cr = pltpu.async_copy(tbl_hbm.at[d], row_v, sem_r)
            if rr == 0:
                ci.wait()
            cr.wait()
            for c in range(n_ch):
                buf = out_bufs[c % 2]
                if pending[c % 2] is not None:
                    pending[c % 2].wait()

                def body(j, carry, _c=c, _buf=buf):
                    idxs = idx_v[pl.ds(_c * B_CH + j * L, L)]
                    _buf[pl.ds(j * L, L)] = plsc.load_gather(row_v, [idxs])
                    return carry

                lax.fori_loop(0, B_CH // L, body, 0, unroll=8)
                pending[c % 2] = pltpu.async_copy(
                    buf, out_hbm.at[d, pl.ds(c * B_CH, B_CH)], out_sems[c % 2])
        for p in pending:
            if p is not None:
                p.wait()

    return gather_k


def _proj_t_body(w_ref, e_ref, b_ref, o_ref):
    o_ref[...] = (
        lax.dot_general(
            w_ref[...],
            e_ref[...],
            (((0,), (0,)), ((), ())),
            preferred_element_type=jnp.float32,
        )
        + b_ref[...]
    )


def _make_tc_proj_t(D, B, blk):
    return pl.pallas_call(
        _proj_t_body,
        grid=(B // blk,),
        in_specs=[
            pl.BlockSpec((D, D), lambda i: (0, 0)),
            pl.BlockSpec((D, blk), lambda i: (0, i)),
            pl.BlockSpec((D, 1), lambda i: (0, 0)),
        ],
        out_specs=pl.BlockSpec((D, blk), lambda i: (0, i)),
        out_shape=jax.ShapeDtypeStruct((D, B), jnp.float32),
    )


def kernel(y, embed_table, W, b):
    V, D = embed_table.shape
    B = y.shape[0]
    tbl_t = embed_table.T
    e_t = _make_sc_gather_t(D, V, B)(tbl_t, y.astype(jnp.int32))
    out_t = _make_tc_proj_t(D, B, blk=4096)(W, e_t, b.reshape(D, 1))
    return out_t.T


# speedup vs baseline: 2.6644x; 1.3771x over previous
import functools

import jax
import jax.numpy as jnp
from jax import lax
from jax.experimental import pallas as pl
from jax.experimental.pallas import tpu as pltpu
from jax.experimental.pallas import tpu_sc as plsc


def _make_sc_gather_t(D, V, B):
    info = plsc.get_sparse_core_info()
    NC, NS, L = info.num_cores, info.num_subcores, info.num_lanes
    NW = NC * NS
    assert D % NW == 0
    rows_per_w = D // NW
    B_CH = 4096 if B % 4096 == 0 else B
    n_ch = B // B_CH
    mesh = plsc.VectorSubcoreMesh(core_axis_name="c", subcore_axis_name="s")

    @functools.partial(
        pl.kernel,
        mesh=mesh,
        out_type=jax.ShapeDtypeStruct((D, B), jnp.float32),
        scratch_types=[
            pltpu.VMEM((V,), jnp.float32),
            pltpu.VMEM((B,), jnp.int32),
            pltpu.VMEM((B_CH,), jnp.float32),
            pltpu.VMEM((B_CH,), jnp.float32),
            pltpu.SemaphoreType.DMA,
            pltpu.SemaphoreType.DMA,
            pltpu.SemaphoreType.DMA,
            pltpu.SemaphoreType.DMA,
        ],
        compiler_params=pltpu.CompilerParams(needs_layout_passes=False),
    )
    def gather_k(tbl_hbm, idx_hbm, out_hbm, row_v, idx_v, out_v0, out_v1,
                 sem_i, sem_r, sem_o0, sem_o1):
        wid = lax.axis_index("s") * NC + lax.axis_index("c")
        out_bufs = (out_v0, out_v1)
        out_sems = (sem_o0, sem_o1)
        pending = [None, None]
        ci = pltpu.async_copy(idx_hbm, idx_v, sem_i)
        for rr in range(rows_per_w):
            d = wid * rows_per_w + rr
            cr = pltpu.async_copy(tbl_hbm.at[d], row_v, sem_r)
            if rr == 0:
                ci.wait()
            cr.wait()
            for c in range(n_ch):
                buf = out_bufs[c % 2]
                if pending[c % 2] is not None:
                    pending[c % 2].wait()

                @plsc.parallel_loop(0, B_CH, L, unroll=8)
                def body(j, _c=c, _buf=buf):
                    idxs = idx_v[pl.ds(_c * B_CH + j, L)]
                    _buf[pl.ds(j, L)] = plsc.load_gather(row_v, [idxs])
                pending[c % 2] = pltpu.async_copy(
                    buf, out_hbm.at[d, pl.ds(c * B_CH, B_CH)], out_sems[c % 2])
        for p in pending:
            if p is not None:
                p.wait()

    return gather_k


def _proj_t_body(w_ref, e_ref, b_ref, o_ref):
    o_ref[...] = (
        lax.dot_general(
            w_ref[...],
            e_ref[...],
            (((0,), (0,)), ((), ())),
            preferred_element_type=jnp.float32,
        )
        + b_ref[...]
    )


def _make_tc_proj_t(D, B, blk):
    return pl.pallas_call(
        _proj_t_body,
        grid=(B // blk,),
        in_specs=[
            pl.BlockSpec((D, D), lambda i: (0, 0)),
            pl.BlockSpec((D, blk), lambda i: (0, i)),
            pl.BlockSpec((D, 1), lambda i: (0, 0)),
        ],
        out_specs=pl.BlockSpec((D, blk), lambda i: (0, i)),
        out_shape=jax.ShapeDtypeStruct((D, B), jnp.float32),
    )


def kernel(y, embed_table, W, b):
    V, D = embed_table.shape
    B = y.shape[0]
    tbl_t = embed_table.T
    e_t = _make_sc_gather_t(D, V, B)(tbl_t, y.astype(jnp.int32))
    out_t = _make_tc_proj_t(D, B, blk=4096)(W, e_t, b.reshape(D, 1))
    return out_t.T
